# baseline (speedup 1.0000x reference)
import jax
import jax.numpy as jnp
from jax.experimental import pallas as pl


def _pick_bm(n):
    for bm in (512, 500, 400, 256, 250, 200, 128, 100, 80, 50, 40, 25, 16, 8):
        if n % bm == 0 and (bm % 8 == 0 or bm == n) and bm * n * 4 <= 17_000_000:
            return bm
    return n


def _deg_kernel(adj_ref, deg_ref):
    deg_ref[:, :] = jnp.sum(adj_ref[:, :], axis=1, keepdims=True) + 1.0


def _prep_kernel(deg_ref, x_ref, w_ref, dis_ref, hp_ref):
    dis = jax.lax.rsqrt(deg_ref[:, :])
    dis = jnp.where(jnp.isinf(dis), 0.0, dis)
    dis_ref[:, :] = dis
    hp_ref[:, :] = dis * jnp.dot(
        x_ref[:, :], w_ref[:, :], preferred_element_type=jnp.float32
    )


def _main_kernel(dis_ref, hp_ref, b_ref, adj_ref, out_ref):
    i = pl.program_id(0)
    bm = out_ref.shape[0]
    acc = jnp.dot(adj_ref[:, :], hp_ref[:, :], preferred_element_type=jnp.float32)
    hp_self = hp_ref[pl.ds(i * bm, bm), :]
    out_ref[:, :] = dis_ref[:, :] * (acc + hp_self) + b_ref[:, :]


def kernel(x, edge_index, edge_weight, W, b):
    n, d_in = x.shape
    d_out = W.shape[1]
    adj = edge_index
    bm = _pick_bm(n)
    nb = n // bm

    deg = pl.pallas_call(
        _deg_kernel,
        grid=(nb,),
        in_specs=[pl.BlockSpec((bm, n), lambda i: (i, 0))],
        out_specs=pl.BlockSpec((bm, 1), lambda i: (i, 0)),
        out_shape=jax.ShapeDtypeStruct((n, 1), jnp.float32),
    )(adj)

    dis, hp = pl.pallas_call(
        _prep_kernel,
        in_specs=[
            pl.BlockSpec((n, 1), lambda: (0, 0)),
            pl.BlockSpec((n, d_in), lambda: (0, 0)),
            pl.BlockSpec((d_in, d_out), lambda: (0, 0)),
        ],
        out_specs=[
            pl.BlockSpec((n, 1), lambda: (0, 0)),
            pl.BlockSpec((n, d_out), lambda: (0, 0)),
        ],
        out_shape=[
            jax.ShapeDtypeStruct((n, 1), jnp.float32),
            jax.ShapeDtypeStruct((n, d_out), jnp.float32),
        ],
    )(deg, x, W)

    out = pl.pallas_call(
        _main_kernel,
        grid=(nb,),
        in_specs=[
            pl.BlockSpec((bm, 1), lambda i: (i, 0)),
            pl.BlockSpec((n, d_out), lambda i: (0, 0)),
            pl.BlockSpec((1, d_out), lambda i: (0, 0)),
            pl.BlockSpec((bm, n), lambda i: (i, 0)),
        ],
        out_specs=pl.BlockSpec((bm, d_out), lambda i: (i, 0)),
        out_shape=jax.ShapeDtypeStruct((n, d_out), jnp.float32),
    )(dis, hp, b.reshape(1, d_out), adj)

    return out


# speedup vs baseline: 1.0594x; 1.0594x over previous
import jax
import jax.numpy as jnp
from jax.experimental import pallas as pl
from jax.experimental.pallas import tpu as pltpu


def _pick_bm(n):
    for bm in (512, 500, 400, 256, 250, 200, 128, 100, 80, 50, 40, 25, 16, 8):
        if n % bm == 0 and (bm % 8 == 0 or bm == n) and bm * n * 4 <= 17_000_000:
            return bm
    return n


def _fused_kernel(x_ref, w_ref, b_ref, adj_ref, out_ref, deg_s, dis_s, hp_s):
    p = pl.program_id(0)
    i = pl.program_id(1)
    nb = pl.num_programs(1)
    bm = out_ref.shape[0]

    @pl.when(p == 0)
    def _phase0():
        deg_s[pl.ds(i * bm, bm), :] = (
            jnp.sum(adj_ref[:, :], axis=1, keepdims=True) + 1.0
        )

    @pl.when((p == 0) & (i == 0))
    def _xw():
        hp_s[:, :] = jnp.dot(
            x_ref[:, :], w_ref[:, :], preferred_element_type=jnp.float32
        )

    @pl.when((p == 0) & (i == nb - 1))
    def _finalize_deg():
        dis = jax.lax.rsqrt(deg_s[:, :])
        dis = jnp.where(jnp.isinf(dis), 0.0, dis)
        dis_s[:, :] = dis
        hp_s[:, :] = dis * hp_s[:, :]

    @pl.when(p == 1)
    def _phase1():
        acc = jnp.dot(
            adj_ref[:, :], hp_s[:, :], preferred_element_type=jnp.float32
        )
        sl = pl.ds(i * bm, bm)
        out_ref[:, :] = dis_s[sl, :] * (acc + hp_s[sl, :]) + b_ref[:, :]


def kernel(x, edge_index, edge_weight, W, b):
    n, d_in = x.shape
    d_out = W.shape[1]
    bm = _pick_bm(n)
    nb = n // bm

    out = pl.pallas_call(
        _fused_kernel,
        grid=(2, nb),
        in_specs=[
            pl.BlockSpec((n, d_in), lambda p, i: (0, 0)),
            pl.BlockSpec((d_in, d_out), lambda p, i: (0, 0)),
            pl.BlockSpec((1, d_out), lambda p, i: (0, 0)),
            pl.BlockSpec((bm, n), lambda p, i: (i, 0)),
        ],
        out_specs=pl.BlockSpec((bm, d_out), lambda p, i: (i * p, 0)),
        out_shape=jax.ShapeDtypeStruct((n, d_out), jnp.float32),
        scratch_shapes=[
            pltpu.VMEM((n, 1), jnp.float32),
            pltpu.VMEM((n, 1), jnp.float32),
            pltpu.VMEM((n, d_out), jnp.float32),
        ],
        compiler_params=pltpu.CompilerParams(
            dimension_semantics=("arbitrary", "arbitrary"),
            vmem_limit_bytes=100 * 1024 * 1024,
        ),
    )(x, W, b.reshape(1, d_out), edge_index)

    return out


# speedup vs baseline: 1.0711x; 1.0110x over previous
import functools

import jax
import jax.numpy as jnp
from jax.experimental import pallas as pl
from jax.experimental.pallas import tpu as pltpu


def _pick_bm(n):
    for bm in (512, 500, 400, 256, 250, 200, 128, 100, 80, 50, 40, 25, 16, 8):
        if n % bm == 0 and (bm % 8 == 0 or bm == n) and bm * n * 4 <= 17_000_000:
            return bm
    return n


def _fused_kernel(nb, x_ref, w_ref, b_ref, adj_ref, out_ref, deg_s, dis_s, hp_s):
    s = pl.program_id(0)
    bm = out_ref.shape[0]

    @pl.when(s < nb)
    def _phase0():
        deg_s[pl.ds(s * bm, bm), :] = (
            jnp.sum(adj_ref[:, :], axis=1, keepdims=True) + 1.0
        )

    @pl.when(s == 0)
    def _xw():
        hp_s[:, :] = jnp.dot(
            x_ref[:, :], w_ref[:, :], preferred_element_type=jnp.float32
        )

    @pl.when(s == nb - 1)
    def _finalize_deg():
        dis = jax.lax.rsqrt(deg_s[:, :])
        dis = jnp.where(jnp.isinf(dis), 0.0, dis)
        dis_s[:, :] = dis
        hp_s[:, :] = dis * hp_s[:, :]

    @pl.when(s >= nb - 1)
    def _phase1():
        blk = jnp.where(s == nb - 1, nb - 1, s - nb)
        acc = jnp.dot(
            adj_ref[:, :], hp_s[:, :], preferred_element_type=jnp.float32
        )
        sl = pl.ds(blk * bm, bm)
        out_ref[:, :] = dis_s[sl, :] * (acc + hp_s[sl, :]) + b_ref[:, :]


def kernel(x, edge_index, edge_weight, W, b):
    n, d_in = x.shape
    d_out = W.shape[1]
    bm = _pick_bm(n)
    nb = n // bm

    out = pl.pallas_call(
        functools.partial(_fused_kernel, nb),
        grid=(2 * nb - 1,),
        in_specs=[
            pl.BlockSpec((n, d_in), lambda s: (0, 0)),
            pl.BlockSpec((d_in, d_out), lambda s: (0, 0)),
            pl.BlockSpec((1, d_out), lambda s: (0, 0)),
            pl.BlockSpec((bm, n), lambda s: (jnp.where(s < nb, s, s - nb), 0)),
        ],
        out_specs=pl.BlockSpec(
            (bm, d_out), lambda s: (jnp.where(s < nb, nb - 1, s - nb), 0)
        ),
        out_shape=jax.ShapeDtypeStruct((n, d_out), jnp.float32),
        scratch_shapes=[
            pltpu.VMEM((n, 1), jnp.float32),
            pltpu.VMEM((n, 1), jnp.float32),
            pltpu.VMEM((n, d_out), jnp.float32),
        ],
        compiler_params=pltpu.CompilerParams(
            dimension_semantics=("arbitrary",),
            vmem_limit_bytes=100 * 1024 * 1024,
        ),
    )(x, W, b.reshape(1, d_out), edge_index)

    return out
